# Initial kernel scaffold; baseline (speedup 1.0000x reference)
import jax
import jax.numpy as jnp
from jax.experimental import pallas as pl


def kernel(x, edge_index, batch, emb_W0, emb_b0, emb_a0, emb_W1, emb_b1, emb_a1, emb_W2, emb_b2, ge_W0, ge_b0, ge_a0, ge_W1, ge_b1, ge_a1, ge_W2, ge_b2, conv0_W, conv0_b, conv1_W, conv1_b, np_W0, np_a0, np_W1, np_a1, np_W2, ep_W0, ep_b0, ep_a0, ep_W1, ep_b1, ep_a1, ep_W2, ep_b2):
    raise NotImplementedError("write your pallas kernel here")


# speedup vs baseline: 4.3614x; 4.3614x over previous
import functools

import jax
import jax.numpy as jnp
from jax import lax
from jax.experimental import pallas as pl
from jax.experimental.pallas import tpu as pltpu
from jax.experimental.pallas import tpu_sc as plsc

N = 10000
NP = 10240
G = 4
E = 320000
EP = 327680
DUMMY = 10016
NC, NS = 2, 16
STRIPE = NP // NS
EPW = EP // (NC * NS)
CH = 128
NCH = EPW // CH
RB = 1024
EB = 4096

_f32 = jnp.float32
_HI = lax.Precision.DEFAULT


def _mm(a, b):
    return jnp.dot(a, b, precision=_HI, preferred_element_type=_f32)


def _prelu(t, a):
    return jnp.where(t >= 0, t, a * t)


def _tc1_body(x_ref, b_ref, W0, b0, a0, W1, b1, a1, W2, b2,
              xemb_ref, sums_ref, cnts_ref):
    i = pl.program_id(0)
    t = _prelu(_mm(x_ref[...], W0[...]) + b0[...], a0[0, 0])
    t = _prelu(_mm(t, W1[...]) + b1[...], a1[0, 0])
    e = _mm(t, W2[...]) + b2[...]
    xemb_ref[...] = e
    oh = (b_ref[...] == lax.broadcasted_iota(jnp.int32, (1, G), 1)).astype(_f32)
    s_blk = lax.dot_general(oh, e, (((0,), (0,)), ((), ())), precision=_HI,
                            preferred_element_type=_f32)
    c_blk = lax.dot_general(oh, jnp.ones((RB, 1), _f32),
                            (((0,), (0,)), ((), ())), precision=_HI,
                            preferred_element_type=_f32)

    @pl.when(i == 0)
    def _():
        sums_ref[...] = jnp.zeros_like(sums_ref)
        cnts_ref[...] = jnp.zeros_like(cnts_ref)

    sums_ref[...] += s_blk
    cnts_ref[...] += c_blk


def _tc2_body(sums_ref, cnts_ref, W0, b0, a0, W1, b1, a1, W2, b2, g_ref):
    g = sums_ref[...] / jnp.maximum(cnts_ref[...], 1.0)
    t = _prelu(_mm(g, W0[...]) + b0[...], a0[0, 0])
    t = _prelu(_mm(t, W1[...]) + b1[...], a1[0, 0])
    g_ref[...] = _mm(t, W2[...]) + b2[...]


def _tc3_body(xemb_ref, b_ref, h0_ref, h1_ref, genc_ref, convW_ref,
              h_ref, xw0s_ref, dinv_ref):
    oh = (b_ref[...] == lax.broadcasted_iota(jnp.int32, (1, G), 1)).astype(_f32)
    h = jnp.concatenate([xemb_ref[...], _mm(oh, genc_ref[...])], axis=1)
    dinv = lax.rsqrt(h0_ref[...] + h1_ref[...] + 1.0)
    h_ref[...] = h
    xw0s_ref[...] = _mm(h, convW_ref[...]) * dinv
    dinv_ref[...] = dinv


def _tc4_body(h_ref, xw0s_ref, dinv_ref, p0_ref, p1_ref, cb_ref, W_ref,
              xw1s_ref):
    dinv = dinv_ref[...]
    gcn0 = dinv * (p0_ref[...] + p1_ref[...] + xw0s_ref[...]) + cb_ref[...]
    h1 = h_ref[...] + jnp.maximum(gcn0, 0.0)
    xw1s_ref[...] = _mm(h1, W_ref[...]) * dinv


def _tc5_body(xemb_ref, xw1s_ref, dinv_ref, p0_ref, p1_ref, cb_ref,
              npW0, npa0, npW1, npa1, npW2, epW0a, epW0b, epb0,
              np_ref, A_ref, B_ref):
    dinv = dinv_ref[...]
    final = xemb_ref[...] + dinv * (p0_ref[...] + p1_ref[...] + xw1s_ref[...]) \
        + cb_ref[...]
    t = _prelu(_mm(final, npW0[...]), npa0[0, 0])
    t = _prelu(_mm(t, npW1[...]), npa1[0, 0])
    np_ref[...] = _mm(t, npW2[...])
    A_ref[...] = _mm(final, epW0a[...]) + epb0[...]
    B_ref[...] = _mm(final, epW0b[...])


def _tc6_body(gA_ref, gB_ref, a0, W1, b1, a1, W2, b2, out_ref):
    e1 = _prelu(gA_ref[...] + gB_ref[...], a0[0, 0])
    e2 = _prelu(_mm(e1, W1[...]) + b1[...], a1[0, 0])
    out_ref[...] = _mm(e2, W2[...]) + b2[...]


def _sc_hist_body(dst_hbm, zeros_hbm, ones_hbm, out_hbm, idx_v, ones_v, acc):
    c = lax.axis_index("c")
    s = lax.axis_index("s")
    stripe = pl.ds(s * STRIPE, STRIPE)
    pltpu.sync_copy(zeros_hbm.at[stripe], acc.at[stripe])
    pltpu.sync_copy(ones_hbm, ones_v)
    plsc.subcore_barrier()
    base_w = (c * NS + s) * EPW

    def chunk(k, _):
        base = base_w + k * CH
        pltpu.sync_copy(dst_hbm.at[pl.ds(base, CH)], idx_v)
        pltpu.sync_copy(ones_v, acc.at[idx_v], add=True)
        return _

    lax.fori_loop(0, NCH, chunk, None)
    plsc.subcore_barrier()
    pltpu.sync_copy(acc.at[stripe], out_hbm.at[c, stripe])


def _sc_mp_body(table_hbm, src_hbm, dst_hbm, zeros_hbm, out_hbm,
                idxs_v, idxd_v, rows_v, acc, sem):
    c = lax.axis_index("c")
    s = lax.axis_index("s")
    stripe = pl.ds(s * STRIPE, STRIPE)
    pltpu.sync_copy(zeros_hbm.at[stripe], acc.at[stripe])
    plsc.subcore_barrier()
    base_w = (c * NS + s) * EPW

    def chunk(k, _):
        base = base_w + k * CH
        pltpu.sync_copy(src_hbm.at[pl.ds(base, CH)], idxs_v)
        pltpu.async_copy(table_hbm.at[idxs_v], rows_v, sem).wait()
        pltpu.sync_copy(dst_hbm.at[pl.ds(base, CH)], idxd_v)
        pltpu.sync_copy(rows_v, acc.at[idxd_v], add=True)
        return _

    lax.fori_loop(0, NCH, chunk, None)
    plsc.subcore_barrier()
    pltpu.sync_copy(acc.at[stripe], out_hbm.at[c, stripe])


def _sc_gather2_body(A_hbm, B_hbm, src_hbm, dst_hbm, gA_hbm, gB_hbm,
                     idx_v, rows_v, sem):
    c = lax.axis_index("c")
    s = lax.axis_index("s")
    base_w = (c * NS + s) * EPW

    def chunk(k, _):
        base = base_w + k * CH
        pltpu.sync_copy(src_hbm.at[pl.ds(base, CH)], idx_v)
        pltpu.async_copy(A_hbm.at[idx_v], rows_v, sem).wait()
        pltpu.sync_copy(rows_v, gA_hbm.at[pl.ds(base, CH)])
        pltpu.sync_copy(dst_hbm.at[pl.ds(base, CH)], idx_v)
        pltpu.async_copy(B_hbm.at[idx_v], rows_v, sem).wait()
        pltpu.sync_copy(rows_v, gB_hbm.at[pl.ds(base, CH)])
        return _

    lax.fori_loop(0, NCH, chunk, None)


@functools.cache
def _sc_mesh():
    return plsc.VectorSubcoreMesh(core_axis_name="c", subcore_axis_name="s",
                                  num_cores=NC, num_subcores=NS)


def _sc_hist(dst_pad, zeros_n, ones_v):
    return pl.kernel(
        _sc_hist_body,
        out_type=jax.ShapeDtypeStruct((NC, NP), _f32),
        mesh=_sc_mesh(),
        compiler_params=pltpu.CompilerParams(use_tc_tiling_on_sc=False),
        scratch_types=[
            pltpu.VMEM((CH,), jnp.int32),
            pltpu.VMEM((CH,), _f32),
            pltpu.VMEM_SHARED((NP,), _f32),
        ],
    )(dst_pad, zeros_n, ones_v)


def _sc_mp(table, src_pad, dst_pad, zeros_nd, d):
    return pl.kernel(
        _sc_mp_body,
        out_type=jax.ShapeDtypeStruct((NC, NP, d), _f32),
        mesh=_sc_mesh(),
        compiler_params=pltpu.CompilerParams(use_tc_tiling_on_sc=False),
        scratch_types=[
            pltpu.VMEM((CH,), jnp.int32),
            pltpu.VMEM((CH,), jnp.int32),
            pltpu.VMEM((CH, d), _f32),
            pltpu.VMEM_SHARED((NP, d), _f32),
            pltpu.SemaphoreType.DMA,
        ],
    )(table, src_pad, dst_pad, zeros_nd)


def _sc_gather2(A, B, src_pad, dst_pad):
    return pl.kernel(
        _sc_gather2_body,
        out_type=(jax.ShapeDtypeStruct((EP, 64), _f32),
                  jax.ShapeDtypeStruct((EP, 64), _f32)),
        mesh=_sc_mesh(),
        compiler_params=pltpu.CompilerParams(use_tc_tiling_on_sc=False),
        scratch_types=[
            pltpu.VMEM((CH,), jnp.int32),
            pltpu.VMEM((CH, 64), _f32),
            pltpu.SemaphoreType.DMA,
        ],
    )(A, B, src_pad, dst_pad)


def _nb(shape):
    return pl.BlockSpec((RB,) + shape[1:], lambda i: (i,) + (0,) * (len(shape) - 1))


def _full(shape):
    return pl.BlockSpec(shape, lambda i: (0,) * len(shape))


_SMEM = pl.BlockSpec(memory_space=pltpu.SMEM)


def _tc1(x_pad, batch_pad, W0, b0, a0, W1, b1, a1, W2, b2):
    return pl.pallas_call(
        _tc1_body,
        grid=(NP // RB,),
        in_specs=[
            _nb((NP, 128)), _nb((NP, 1)),
            _full((128, 128)), _full((1, 128)), _SMEM,
            _full((128, 128)), _full((1, 128)), _SMEM,
            _full((128, 64)), _full((1, 64)),
        ],
        out_specs=[_nb((NP, 64)), _full((G, 64)), _full((G, 1))],
        out_shape=[jax.ShapeDtypeStruct((NP, 64), _f32),
                   jax.ShapeDtypeStruct((G, 64), _f32),
                   jax.ShapeDtypeStruct((G, 1), _f32)],
    )(x_pad, batch_pad, W0, b0, a0, W1, b1, a1, W2, b2)


def _tc2(sums, cnts, W0, b0, a0, W1, b1, a1, W2, b2):
    return pl.pallas_call(
        _tc2_body,
        in_specs=[pl.BlockSpec((G, 64)), pl.BlockSpec((G, 1)),
                  pl.BlockSpec((64, 128)), pl.BlockSpec((1, 128)), _SMEM,
                  pl.BlockSpec((128, 128)), pl.BlockSpec((1, 128)), _SMEM,
                  pl.BlockSpec((128, 64)), pl.BlockSpec((1, 64))],
        out_shape=jax.ShapeDtypeStruct((G, 64), _f32),
    )(sums, cnts, W0, b0, a0, W1, b1, a1, W2, b2)


def _tc3(x_emb, batch_pad, h0, h1, g_enc, conv0_W):
    return pl.pallas_call(
        _tc3_body,
        grid=(NP // RB,),
        in_specs=[_nb((NP, 64)), _nb((NP, 1)), _nb((NP, 1)), _nb((NP, 1)),
                  _full((G, 64)), _full((128, 128))],
        out_specs=[_nb((NP, 128)), _nb((NP, 128)), _nb((NP, 1))],
        out_shape=[jax.ShapeDtypeStruct((NP, 128), _f32),
                   jax.ShapeDtypeStruct((NP, 128), _f32),
                   jax.ShapeDtypeStruct((NP, 1), _f32)],
    )(x_emb, batch_pad, h0, h1, g_enc, conv0_W)


def _tc4(h, xw0s, dinv, p0, p1, conv0_b, conv1_W):
    return pl.pallas_call(
        _tc4_body,
        grid=(NP // RB,),
        in_specs=[_nb((NP, 128)), _nb((NP, 128)), _nb((NP, 1)),
                  _nb((NP, 128)), _nb((NP, 128)),
                  _full((1, 128)), _full((128, 64))],
        out_specs=[_nb((NP, 64))],
        out_shape=[jax.ShapeDtypeStruct((NP, 64), _f32)],
    )(h, xw0s, dinv, p0, p1, conv0_b, conv1_W)[0]


def _tc5(x_emb, xw1s, dinv, p0, p1, conv1_b,
         npW0, npa0, npW1, npa1, npW2, epW0a, epW0b, epb0):
    return pl.pallas_call(
        _tc5_body,
        grid=(NP // RB,),
        in_specs=[_nb((NP, 64)), _nb((NP, 64)), _nb((NP, 1)),
                  _nb((NP, 64)), _nb((NP, 64)), _full((1, 64)),
                  _full((64, 64)), _SMEM, _full((64, 64)), _SMEM,
                  _full((64, 8)), _full((64, 64)), _full((64, 64)),
                  _full((1, 64))],
        out_specs=[_nb((NP, 8)), _nb((NP, 64)), _nb((NP, 64))],
        out_shape=[jax.ShapeDtypeStruct((NP, 8), _f32),
                   jax.ShapeDtypeStruct((NP, 64), _f32),
                   jax.ShapeDtypeStruct((NP, 64), _f32)],
    )(x_emb, xw1s, dinv, p0, p1, conv1_b,
      npW0, npa0, npW1, npa1, npW2, epW0a, epW0b, epb0)


def _eb(shape):
    return pl.BlockSpec((EB,) + shape[1:], lambda i: (i,) + (0,) * (len(shape) - 1))


def _tc6(gA, gB, a0, W1, b1, a1, W2, b2):
    return pl.pallas_call(
        _tc6_body,
        grid=(EP // EB,),
        in_specs=[_eb((EP, 64)), _eb((EP, 64)), _SMEM,
                  _full((64, 64)), _full((1, 64)), _SMEM,
                  _full((64, 4)), _full((1, 4))],
        out_specs=[_eb((EP, 4))],
        out_shape=[jax.ShapeDtypeStruct((EP, 4), _f32)],
    )(gA, gB, a0, W1, b1, a1, W2, b2)[0]


def kernel(x, edge_index, batch, emb_W0, emb_b0, emb_a0, emb_W1, emb_b1,
           emb_a1, emb_W2, emb_b2, ge_W0, ge_b0, ge_a0, ge_W1, ge_b1, ge_a1,
           ge_W2, ge_b2, conv0_W, conv0_b, conv1_W, conv1_b, np_W0, np_a0,
           np_W1, np_a1, np_W2, ep_W0, ep_b0, ep_a0, ep_W1, ep_b1, ep_a1,
           ep_W2, ep_b2):
    r1 = lambda v: v.reshape(1, -1)
    s11 = lambda v: v.reshape(1, 1)

    src = edge_index[0].astype(jnp.int32)
    dst = edge_index[1].astype(jnp.int32)
    epad = jnp.full((EP - E,), DUMMY, jnp.int32)
    src_pad = jnp.concatenate([src, epad])
    dst_pad = jnp.concatenate([dst, epad])
    x_pad = jnp.concatenate([x, jnp.zeros((NP - N, 128), _f32)])
    batch_pad = jnp.concatenate(
        [batch.astype(jnp.int32), jnp.full((NP - N,), G, jnp.int32)]
    ).reshape(NP, 1)

    zeros_n = jnp.zeros((NP,), _f32)
    ones_v = jnp.ones((CH,), _f32)

    hist = _sc_hist(dst_pad, zeros_n, ones_v)

    x_emb, sums, cnts = _tc1(x_pad, batch_pad,
                             emb_W0, r1(emb_b0), s11(emb_a0),
                             emb_W1, r1(emb_b1), s11(emb_a1),
                             emb_W2, r1(emb_b2))
    g_enc = _tc2(sums, cnts, ge_W0, r1(ge_b0), s11(ge_a0),
                 ge_W1, r1(ge_b1), s11(ge_a1), ge_W2, r1(ge_b2))

    h, xw0s, dinv = _tc3(x_emb, batch_pad,
                         hist[0].reshape(NP, 1), hist[1].reshape(NP, 1),
                         g_enc, conv0_W)

    P0 = _sc_mp(xw0s, src_pad, dst_pad, jnp.zeros((NP, 128), _f32), 128)
    xw1s = _tc4(h, xw0s, dinv, P0[0], P0[1], r1(conv0_b), conv1_W)

    P1 = _sc_mp(xw1s, src_pad, dst_pad, jnp.zeros((NP, 64), _f32), 64)
    node_preds, A, B = _tc5(x_emb, xw1s, dinv, P1[0], P1[1], r1(conv1_b),
                            np_W0, s11(np_a0), np_W1, s11(np_a1), np_W2,
                            ep_W0[:64], ep_W0[64:], r1(ep_b0))

    gA, gB = _sc_gather2(A, B, src_pad, dst_pad)
    edge_preds = _tc6(gA, gB, s11(ep_a0), ep_W1, r1(ep_b1), s11(ep_a1),
                      ep_W2, r1(ep_b2))

    return (node_preds[:N], edge_preds[:E])


# speedup vs baseline: 5.6221x; 1.2890x over previous
import functools

import jax
import jax.numpy as jnp
from jax import lax
from jax.experimental import pallas as pl
from jax.experimental.pallas import tpu as pltpu
from jax.experimental.pallas import tpu_sc as plsc

N = 10000
NP = 10240
G = 4
E = 320000
EP = 327680
DUMMY = 10016
NC, NS = 2, 16
STRIPE = NP // NS
EPW = EP // (NC * NS)
CH = 128
NCH = EPW // CH
RB = 1024
EB = 4096

_f32 = jnp.float32
_HI = lax.Precision.DEFAULT


def _mm(a, b):
    return jnp.dot(a, b, precision=_HI, preferred_element_type=_f32)


def _prelu(t, a):
    return jnp.where(t >= 0, t, a * t)


def _tc1_body(x_ref, b_ref, W0, b0, a0, W1, b1, a1, W2, b2,
              xemb_ref, sums_ref, cnts_ref):
    i = pl.program_id(0)
    t = _prelu(_mm(x_ref[...], W0[...]) + b0[...], a0[0, 0])
    t = _prelu(_mm(t, W1[...]) + b1[...], a1[0, 0])
    e = _mm(t, W2[...]) + b2[...]
    xemb_ref[...] = e
    oh = (b_ref[...] == lax.broadcasted_iota(jnp.int32, (1, G), 1)).astype(_f32)
    s_blk = lax.dot_general(oh, e, (((0,), (0,)), ((), ())), precision=_HI,
                            preferred_element_type=_f32)
    c_blk = lax.dot_general(oh, jnp.ones((RB, 1), _f32),
                            (((0,), (0,)), ((), ())), precision=_HI,
                            preferred_element_type=_f32)

    @pl.when(i == 0)
    def _():
        sums_ref[...] = jnp.zeros_like(sums_ref)
        cnts_ref[...] = jnp.zeros_like(cnts_ref)

    sums_ref[...] += s_blk
    cnts_ref[...] += c_blk


def _tc2_body(sums_ref, cnts_ref, W0, b0, a0, W1, b1, a1, W2, b2, g_ref):
    g = sums_ref[...] / jnp.maximum(cnts_ref[...], 1.0)
    t = _prelu(_mm(g, W0[...]) + b0[...], a0[0, 0])
    t = _prelu(_mm(t, W1[...]) + b1[...], a1[0, 0])
    g_ref[...] = _mm(t, W2[...]) + b2[...]


def _tc3_body(xemb_ref, b_ref, h0_ref, h1_ref, genc_ref, convW_ref,
              h_ref, xw0s_ref, dinv_ref):
    oh = (b_ref[...] == lax.broadcasted_iota(jnp.int32, (1, G), 1)).astype(_f32)
    h = jnp.concatenate([xemb_ref[...], _mm(oh, genc_ref[...])], axis=1)
    dinv = lax.rsqrt(h0_ref[...] + h1_ref[...] + 1.0)
    h_ref[...] = h
    xw0s_ref[...] = _mm(h, convW_ref[...]) * dinv
    dinv_ref[...] = dinv


def _tc4_body(h_ref, xw0s_ref, dinv_ref, p0_ref, p1_ref, cb_ref, W_ref,
              xw1s_ref):
    dinv = dinv_ref[...]
    gcn0 = dinv * (p0_ref[...] + p1_ref[...] + xw0s_ref[...]) + cb_ref[...]
    h1 = h_ref[...] + jnp.maximum(gcn0, 0.0)
    xw1s_ref[...] = _mm(h1, W_ref[...]) * dinv


def _tc5_body(xemb_ref, xw1s_ref, dinv_ref, p0_ref, p1_ref, cb_ref,
              npW0, npa0, npW1, npa1, npW2, epW0a, epW0b, epb0,
              np_ref, A_ref, B_ref):
    dinv = dinv_ref[...]
    final = xemb_ref[...] + dinv * (p0_ref[...] + p1_ref[...] + xw1s_ref[...]) \
        + cb_ref[...]
    t = _prelu(_mm(final, npW0[...]), npa0[0, 0])
    t = _prelu(_mm(t, npW1[...]), npa1[0, 0])
    np_ref[...] = _mm(t, npW2[...])
    A_ref[...] = _mm(final, epW0a[...]) + epb0[...]
    B_ref[...] = _mm(final, epW0b[...])


def _tc6_body(gA_ref, gB_ref, a0, W1, b1, a1, W2, b2, out_ref):
    e1 = _prelu(gA_ref[...] + gB_ref[...], a0[0, 0])
    e2 = _prelu(_mm(e1, W1[...]) + b1[...], a1[0, 0])
    out_ref[...] = _mm(e2, W2[...]) + b2[...]


def _sc_hist_body(dst_hbm, zeros_hbm, ones_hbm, out_hbm, dsts_v, ones_v, acc):
    c = lax.axis_index("c")
    s = lax.axis_index("s")
    stripe = pl.ds(s * STRIPE, STRIPE)
    w = c * NS + s
    pltpu.sync_copy(zeros_hbm.at[stripe], acc.at[stripe])
    pltpu.sync_copy(ones_hbm, ones_v)
    pltpu.sync_copy(dst_hbm.at[pl.ds(w * NCH, NCH)], dsts_v)
    plsc.subcore_barrier()

    def chunk(k, _):
        pltpu.sync_copy(ones_v, acc.at[dsts_v.at[k]], add=True)
        return _

    lax.fori_loop(0, NCH, chunk, None)
    plsc.subcore_barrier()
    pltpu.sync_copy(acc.at[stripe], out_hbm.at[c, stripe])


def _sc_mp_body(table_hbm, src_hbm, dst_hbm, zeros_hbm, out_hbm,
                i0, i1, dsts_v, buf0, buf1, acc,
                semg0, semg1, semi0, semi1):
    c = lax.axis_index("c")
    s = lax.axis_index("s")
    stripe = pl.ds(s * STRIPE, STRIPE)
    w = c * NS + s
    row0 = w * NCH
    pltpu.sync_copy(zeros_hbm.at[stripe], acc.at[stripe])
    pltpu.sync_copy(dst_hbm.at[pl.ds(row0, NCH)], dsts_v)
    plsc.subcore_barrier()

    def wait_g(sem, buf):
        pltpu.make_async_copy(table_hbm.at[i0.at[0]], buf, sem).wait()

    def wait_i(sem, ibuf):
        pltpu.make_async_copy(src_hbm.at[pl.ds(row0, 1)], ibuf, sem).wait()

    pltpu.sync_copy(src_hbm.at[pl.ds(row0, 1)], i0)
    pltpu.async_copy(src_hbm.at[pl.ds(row0 + 1, 1)], i1, semi1)
    pltpu.async_copy(table_hbm.at[i0.at[0]], buf0, semg0)

    def body(i, _):
        k0 = 2 * i
        k1 = k0 + 1
        wait_g(semg0, buf0)
        wait_i(semi1, i1)

        @pl.when(i < NCH // 2 - 1)
        def _():
            pltpu.async_copy(src_hbm.at[pl.ds(row0 + k0 + 2, 1)], i0, semi0)

        pltpu.async_copy(table_hbm.at[i1.at[0]], buf1, semg1)
        pltpu.sync_copy(buf0, acc.at[dsts_v.at[k0]], add=True)
        wait_g(semg1, buf1)

        @pl.when(i < NCH // 2 - 1)
        def _():
            wait_i(semi0, i0)
            pltpu.async_copy(src_hbm.at[pl.ds(row0 + k1 + 2, 1)], i1, semi1)
            pltpu.async_copy(table_hbm.at[i0.at[0]], buf0, semg0)

        pltpu.sync_copy(buf1, acc.at[dsts_v.at[k1]], add=True)
        return _

    lax.fori_loop(0, NCH // 2, body, None)
    plsc.subcore_barrier()
    pltpu.sync_copy(acc.at[stripe], out_hbm.at[c, stripe])


def _sc_gather2_body(A_hbm, B_hbm, src_hbm, dst_hbm, gA_hbm, gB_hbm,
                     srcs_v, dsts_v, a0, b0, a1, b1,
                     semg0, semg1, semw0, semw1):
    c = lax.axis_index("c")
    s = lax.axis_index("s")
    w = c * NS + s
    base_w = w * EPW
    pltpu.sync_copy(src_hbm.at[pl.ds(w * NCH, NCH)], srcs_v)
    pltpu.sync_copy(dst_hbm.at[pl.ds(w * NCH, NCH)], dsts_v)

    def wait_w(sem, buf):
        pltpu.make_async_copy(buf, gA_hbm.at[pl.ds(base_w, CH)], sem).wait()

    def wait_g(sem, buf):
        pltpu.make_async_copy(A_hbm.at[srcs_v.at[0]], buf, sem).wait()

    pltpu.async_copy(A_hbm.at[srcs_v.at[0]], a0, semg0)
    pltpu.async_copy(B_hbm.at[dsts_v.at[0]], b0, semg0)

    def body(i, _):
        k0 = 2 * i
        k1 = k0 + 1
        out0 = pl.ds(base_w + k0 * CH, CH)
        out1 = pl.ds(base_w + k1 * CH, CH)
        wait_g(semg0, a0)
        wait_g(semg0, b0)

        @pl.when(i > 0)
        def _():
            wait_w(semw1, a1)
            wait_w(semw1, b1)

        pltpu.async_copy(A_hbm.at[srcs_v.at[k1]], a1, semg1)
        pltpu.async_copy(B_hbm.at[dsts_v.at[k1]], b1, semg1)
        pltpu.async_copy(a0, gA_hbm.at[out0], semw0)
        pltpu.async_copy(b0, gB_hbm.at[out0], semw0)
        wait_g(semg1, a1)
        wait_g(semg1, b1)

        @pl.when(i < NCH // 2 - 1)
        def _():
            wait_w(semw0, a0)
            wait_w(semw0, b0)
            pltpu.async_copy(A_hbm.at[srcs_v.at[k0 + 2]], a0, semg0)
            pltpu.async_copy(B_hbm.at[dsts_v.at[k0 + 2]], b0, semg0)

        pltpu.async_copy(a1, gA_hbm.at[out1], semw1)
        pltpu.async_copy(b1, gB_hbm.at[out1], semw1)
        return _

    lax.fori_loop(0, NCH // 2, body, None)
    wait_w(semw0, a0)
    wait_w(semw0, b0)
    wait_w(semw1, a1)
    wait_w(semw1, b1)


@functools.cache
def _sc_mesh():
    return plsc.VectorSubcoreMesh(core_axis_name="c", subcore_axis_name="s",
                                  num_cores=NC, num_subcores=NS)


def _sc_hist(dst_pad, zeros_n, ones_v):
    return pl.kernel(
        _sc_hist_body,
        out_type=jax.ShapeDtypeStruct((NC, NP), _f32),
        mesh=_sc_mesh(),
        compiler_params=pltpu.CompilerParams(use_tc_tiling_on_sc=False),
        scratch_types=[
            pltpu.VMEM((NCH, CH), jnp.int32),
            pltpu.VMEM((CH,), _f32),
            pltpu.VMEM_SHARED((NP,), _f32),
        ],
    )(dst_pad, zeros_n, ones_v)


def _sc_mp(table, src_pad, dst_pad, zeros_nd, d):
    return pl.kernel(
        _sc_mp_body,
        out_type=jax.ShapeDtypeStruct((NC, NP, d), _f32),
        mesh=_sc_mesh(),
        compiler_params=pltpu.CompilerParams(use_tc_tiling_on_sc=False),
        scratch_types=[
            pltpu.VMEM((1, CH), jnp.int32),
            pltpu.VMEM((1, CH), jnp.int32),
            pltpu.VMEM((NCH, CH), jnp.int32),
            pltpu.VMEM((CH, d), _f32),
            pltpu.VMEM((CH, d), _f32),
            pltpu.VMEM_SHARED((NP, d), _f32),
            pltpu.SemaphoreType.DMA,
            pltpu.SemaphoreType.DMA,
            pltpu.SemaphoreType.DMA,
            pltpu.SemaphoreType.DMA,
        ],
    )(table, src_pad, dst_pad, zeros_nd)


def _sc_gather2(A, B, src_pad, dst_pad):
    return pl.kernel(
        _sc_gather2_body,
        out_type=(jax.ShapeDtypeStruct((EP, 64), _f32),
                  jax.ShapeDtypeStruct((EP, 64), _f32)),
        mesh=_sc_mesh(),
        compiler_params=pltpu.CompilerParams(use_tc_tiling_on_sc=False),
        scratch_types=[
            pltpu.VMEM((NCH, CH), jnp.int32),
            pltpu.VMEM((NCH, CH), jnp.int32),
            pltpu.VMEM((CH, 64), _f32),
            pltpu.VMEM((CH, 64), _f32),
            pltpu.VMEM((CH, 64), _f32),
            pltpu.VMEM((CH, 64), _f32),
            pltpu.SemaphoreType.DMA,
            pltpu.SemaphoreType.DMA,
            pltpu.SemaphoreType.DMA,
            pltpu.SemaphoreType.DMA,
        ],
    )(A, B, src_pad, dst_pad)


def _nb(shape):
    return pl.BlockSpec((RB,) + shape[1:], lambda i: (i,) + (0,) * (len(shape) - 1))


def _full(shape):
    return pl.BlockSpec(shape, lambda i: (0,) * len(shape))


_SMEM = pl.BlockSpec(memory_space=pltpu.SMEM)


def _tc1(x_pad, batch_pad, W0, b0, a0, W1, b1, a1, W2, b2):
    return pl.pallas_call(
        _tc1_body,
        grid=(NP // RB,),
        in_specs=[
            _nb((NP, 128)), _nb((NP, 1)),
            _full((128, 128)), _full((1, 128)), _SMEM,
            _full((128, 128)), _full((1, 128)), _SMEM,
            _full((128, 64)), _full((1, 64)),
        ],
        out_specs=[_nb((NP, 64)), _full((G, 64)), _full((G, 1))],
        out_shape=[jax.ShapeDtypeStruct((NP, 64), _f32),
                   jax.ShapeDtypeStruct((G, 64), _f32),
                   jax.ShapeDtypeStruct((G, 1), _f32)],
    )(x_pad, batch_pad, W0, b0, a0, W1, b1, a1, W2, b2)


def _tc2(sums, cnts, W0, b0, a0, W1, b1, a1, W2, b2):
    return pl.pallas_call(
        _tc2_body,
        in_specs=[pl.BlockSpec((G, 64)), pl.BlockSpec((G, 1)),
                  pl.BlockSpec((64, 128)), pl.BlockSpec((1, 128)), _SMEM,
                  pl.BlockSpec((128, 128)), pl.BlockSpec((1, 128)), _SMEM,
                  pl.BlockSpec((128, 64)), pl.BlockSpec((1, 64))],
        out_shape=jax.ShapeDtypeStruct((G, 64), _f32),
    )(sums, cnts, W0, b0, a0, W1, b1, a1, W2, b2)


def _tc3(x_emb, batch_pad, h0, h1, g_enc, conv0_W):
    return pl.pallas_call(
        _tc3_body,
        grid=(NP // RB,),
        in_specs=[_nb((NP, 64)), _nb((NP, 1)), _nb((NP, 1)), _nb((NP, 1)),
                  _full((G, 64)), _full((128, 128))],
        out_specs=[_nb((NP, 128)), _nb((NP, 128)), _nb((NP, 1))],
        out_shape=[jax.ShapeDtypeStruct((NP, 128), _f32),
                   jax.ShapeDtypeStruct((NP, 128), _f32),
                   jax.ShapeDtypeStruct((NP, 1), _f32)],
    )(x_emb, batch_pad, h0, h1, g_enc, conv0_W)


def _tc4(h, xw0s, dinv, p0, p1, conv0_b, conv1_W):
    return pl.pallas_call(
        _tc4_body,
        grid=(NP // RB,),
        in_specs=[_nb((NP, 128)), _nb((NP, 128)), _nb((NP, 1)),
                  _nb((NP, 128)), _nb((NP, 128)),
                  _full((1, 128)), _full((128, 64))],
        out_specs=[_nb((NP, 64))],
        out_shape=[jax.ShapeDtypeStruct((NP, 64), _f32)],
    )(h, xw0s, dinv, p0, p1, conv0_b, conv1_W)[0]


def _tc5(x_emb, xw1s, dinv, p0, p1, conv1_b,
         npW0, npa0, npW1, npa1, npW2, epW0a, epW0b, epb0):
    return pl.pallas_call(
        _tc5_body,
        grid=(NP // RB,),
        in_specs=[_nb((NP, 64)), _nb((NP, 64)), _nb((NP, 1)),
                  _nb((NP, 64)), _nb((NP, 64)), _full((1, 64)),
                  _full((64, 64)), _SMEM, _full((64, 64)), _SMEM,
                  _full((64, 8)), _full((64, 64)), _full((64, 64)),
                  _full((1, 64))],
        out_specs=[_nb((NP, 8)), _nb((NP, 64)), _nb((NP, 64))],
        out_shape=[jax.ShapeDtypeStruct((NP, 8), _f32),
                   jax.ShapeDtypeStruct((NP, 64), _f32),
                   jax.ShapeDtypeStruct((NP, 64), _f32)],
    )(x_emb, xw1s, dinv, p0, p1, conv1_b,
      npW0, npa0, npW1, npa1, npW2, epW0a, epW0b, epb0)


def _eb(shape):
    return pl.BlockSpec((EB,) + shape[1:], lambda i: (i,) + (0,) * (len(shape) - 1))


def _tc6(gA, gB, a0, W1, b1, a1, W2, b2):
    return pl.pallas_call(
        _tc6_body,
        grid=(EP // EB,),
        in_specs=[_eb((EP, 64)), _eb((EP, 64)), _SMEM,
                  _full((64, 64)), _full((1, 64)), _SMEM,
                  _full((64, 4)), _full((1, 4))],
        out_specs=[_eb((EP, 4))],
        out_shape=[jax.ShapeDtypeStruct((EP, 4), _f32)],
    )(gA, gB, a0, W1, b1, a1, W2, b2)[0]


def kernel(x, edge_index, batch, emb_W0, emb_b0, emb_a0, emb_W1, emb_b1,
           emb_a1, emb_W2, emb_b2, ge_W0, ge_b0, ge_a0, ge_W1, ge_b1, ge_a1,
           ge_W2, ge_b2, conv0_W, conv0_b, conv1_W, conv1_b, np_W0, np_a0,
           np_W1, np_a1, np_W2, ep_W0, ep_b0, ep_a0, ep_W1, ep_b1, ep_a1,
           ep_W2, ep_b2):
    r1 = lambda v: v.reshape(1, -1)
    s11 = lambda v: v.reshape(1, 1)

    src = edge_index[0].astype(jnp.int32)
    dst = edge_index[1].astype(jnp.int32)
    epad = jnp.full((EP - E,), DUMMY, jnp.int32)
    src_pad = jnp.concatenate([src, epad]).reshape(EP // CH, CH)
    dst_pad = jnp.concatenate([dst, epad]).reshape(EP // CH, CH)
    x_pad = jnp.concatenate([x, jnp.zeros((NP - N, 128), _f32)])
    batch_pad = jnp.concatenate(
        [batch.astype(jnp.int32), jnp.full((NP - N,), G, jnp.int32)]
    ).reshape(NP, 1)

    zeros_n = jnp.zeros((NP,), _f32)
    ones_v = jnp.ones((CH,), _f32)

    hist = _sc_hist(dst_pad, zeros_n, ones_v)

    x_emb, sums, cnts = _tc1(x_pad, batch_pad,
                             emb_W0, r1(emb_b0), s11(emb_a0),
                             emb_W1, r1(emb_b1), s11(emb_a1),
                             emb_W2, r1(emb_b2))
    g_enc = _tc2(sums, cnts, ge_W0, r1(ge_b0), s11(ge_a0),
                 ge_W1, r1(ge_b1), s11(ge_a1), ge_W2, r1(ge_b2))

    h, xw0s, dinv = _tc3(x_emb, batch_pad,
                         hist[0].reshape(NP, 1), hist[1].reshape(NP, 1),
                         g_enc, conv0_W)

    P0 = _sc_mp(xw0s, src_pad, dst_pad, jnp.zeros((NP, 128), _f32), 128)
    xw1s = _tc4(h, xw0s, dinv, P0[0], P0[1], r1(conv0_b), conv1_W)

    P1 = _sc_mp(xw1s, src_pad, dst_pad, jnp.zeros((NP, 64), _f32), 64)
    node_preds, A, B = _tc5(x_emb, xw1s, dinv, P1[0], P1[1], r1(conv1_b),
                            np_W0, s11(np_a0), np_W1, s11(np_a1), np_W2,
                            ep_W0[:64], ep_W0[64:], r1(ep_b0))

    gA, gB = _sc_gather2(A, B, src_pad, dst_pad)
    edge_preds = _tc6(gA, gB, s11(ep_a0), ep_W1, r1(ep_b1), s11(ep_a1),
                      ep_W2, r1(ep_b2))

    return (node_preds[:N], edge_preds[:E])
